# baseline (speedup 1.0000x reference)
import jax
import jax.numpy as jnp
from jax import lax
from jax.experimental import pallas as pl
from jax.experimental.pallas import tpu as pltpu
from jax.experimental.pallas import tpu_sc as plsc

_NCLS = 80
_CONF = 0.05
_THR = 0.45
_NPAD = 5120
_CHUNK = 640
_SEG = 128
_B = 4


def _prep_body(pred_ref, boxes_ref, ext_ref, maxc_ref):
    p = pred_ref[0]
    cx = p[:, 0:1]
    cy = p[:, 1:2]
    w = p[:, 2:3]
    h = p[:, 3:4]
    x1 = cx - w / 2.0
    y1 = cy - h / 2.0
    x2 = cx + w / 2.0
    y2 = cy + h / 2.0
    boxes_ref[0, :, 0:1] = x1
    boxes_ref[0, :, 1:2] = y1
    boxes_ref[0, :, 2:3] = x2
    boxes_ref[0, :, 3:4] = y2
    obj = p[:, 4:5]
    cl = p[:, 5:5 + _NCLS]
    cconf = jnp.max(cl, axis=1, keepdims=True)
    li = lax.broadcasted_iota(jnp.int32, cl.shape, 1)
    cpred = jnp.min(jnp.where(cl == cconf, li, _NCLS), axis=1, keepdims=True)
    score = obj * cconf
    mask = score >= _CONF
    ceff = jnp.where(mask, cpred, 127)
    ext_ref[0, :, 0:1] = obj
    ext_ref[0, :, 1:2] = cconf
    ext_ref[0, :, 2:3] = cpred.astype(jnp.float32)
    ext_ref[0, :, 3:4] = mask.astype(jnp.float32)
    ext_ref[0, :, 4:5] = ceff.astype(jnp.float32)
    ext_ref[0, :, 5:6] = score
    m = jnp.maximum(jnp.maximum(jnp.max(x1), jnp.max(y1)),
                    jnp.maximum(jnp.max(x2), jnp.max(y2)))
    c = pl.program_id(1)

    @pl.when(c == 0)
    def _():
        maxc_ref[0] = jnp.full((1, 128), m, jnp.float32)

    @pl.when(c > 0)
    def _():
        maxc_ref[0] = jnp.maximum(maxc_ref[0], m)


def _prep(pred):
    b = pred.shape[0]
    nchunks = _NPAD // _CHUNK
    return pl.pallas_call(
        _prep_body,
        grid=(b, nchunks),
        in_specs=[pl.BlockSpec((1, _CHUNK, 5 + _NCLS), lambda i, c: (i, c, 0))],
        out_specs=[
            pl.BlockSpec((1, _CHUNK, 4), lambda i, c: (i, c, 0)),
            pl.BlockSpec((1, _CHUNK, 8), lambda i, c: (i, c, 0)),
            pl.BlockSpec((1, 1, 128), lambda i, c: (i, 0, 0)),
        ],
        out_shape=[
            jax.ShapeDtypeStruct((b, _NPAD, 4), jnp.float32),
            jax.ShapeDtypeStruct((b, _NPAD, 8), jnp.float32),
            jax.ShapeDtypeStruct((b, 1, 128), jnp.float32),
        ],
    )(pred)


def _offtab_body(boxes_ref, ext_ref, maxb_ref, tab_ref, seg_ref):
    mc = maxb_ref[0, 0, 0] + 1.0
    off = ext_ref[0, :, 2:3] * mc
    x1 = boxes_ref[0, :, 0:1] + off
    y1 = boxes_ref[0, :, 1:2] + off
    x2 = boxes_ref[0, :, 2:3] + off
    y2 = boxes_ref[0, :, 3:4] + off
    tab_ref[0, :, 0:1] = x1
    tab_ref[0, :, 1:2] = y1
    tab_ref[0, :, 2:3] = x2
    tab_ref[0, :, 3:4] = y2
    tab_ref[0, :, 4:5] = (jnp.maximum(x2 - x1, 0.0)
                          * jnp.maximum(y2 - y1, 0.0))
    ceff = ext_ref[0, :, 4:5]
    thr = lax.broadcasted_iota(jnp.int32, (1, _SEG), 1).astype(jnp.float32)
    cnt = jnp.sum((ceff < thr).astype(jnp.int32), axis=0, keepdims=True)
    c = pl.program_id(1)

    @pl.when(c == 0)
    def _():
        seg_ref[0] = cnt

    @pl.when(c > 0)
    def _():
        seg_ref[0] = seg_ref[0] + cnt


def _offtab(boxes, ext, maxb):
    b = boxes.shape[0]
    nchunks = _NPAD // _CHUNK
    return pl.pallas_call(
        _offtab_body,
        grid=(b, nchunks),
        in_specs=[
            pl.BlockSpec((1, _CHUNK, 4), lambda i, c: (i, c, 0)),
            pl.BlockSpec((1, _CHUNK, 8), lambda i, c: (i, c, 0)),
            pl.BlockSpec((1, 1, 1), lambda i, c: (i, 0, 0),
                         memory_space=pltpu.SMEM),
        ],
        out_specs=[
            pl.BlockSpec((1, _CHUNK, 5), lambda i, c: (i, c, 0)),
            pl.BlockSpec((1, 1, _SEG), lambda i, c: (i, 0, 0)),
        ],
        out_shape=[
            jax.ShapeDtypeStruct((b, _NPAD, 5), jnp.float32),
            jax.ShapeDtypeStruct((b, 1, _SEG), jnp.int32),
        ],
    )(boxes, ext, maxb)


def _sc_body(tab_hbm, gidx_hbm, seg_hbm, out_hbm,
             tab_v, gidx_v, seg_v, sup_v, oidx_v, sem):
    ci = lax.axis_index("c")
    si = lax.axis_index("s")
    img = ci * 2 + si // 8
    slot = si % 8
    pltpu.sync_copy(tab_hbm.at[img], tab_v.at[pl.ds(0, _NPAD * 5)])
    pltpu.sync_copy(gidx_hbm.at[img], gidx_v.at[pl.ds(0, _NPAD)])
    pltpu.sync_copy(seg_hbm.at[img], seg_v.at[pl.ds(0, _SEG)])
    iota = lax.iota(jnp.int32, 16)
    base_out = (slot * _B + img) * _NPAD

    def init_body(k, _):
        b0 = k * 16
        sup_v[pl.ds(b0, 16)] = jnp.zeros((16,), jnp.float32)
        oidx_v[k // 8, pl.ds((k % 8) * 16, 16)] = (
            gidx_v[pl.ds(b0, 16)] + base_out)
        return 0

    lax.fori_loop(0, _NPAD // 16, init_body, 0)

    def tabcol(col, idx16):
        return plsc.load_gather(tab_v, [idx16 * 5 + col])

    def do_class(c):
        s = seg_v[pl.ds(c, 16)][0]
        e = seg_v[pl.ds(c + 1, 16)][0]

        def i_body(i, _):
            supi = sup_v[pl.ds(i, 16)][0]

            @pl.when(supi < 0.5)
            def _():
                oi = gidx_v[pl.ds(i, 16)][0]
                rowi = tab_v[pl.ds(oi * 5, 16)]
                x1i = jnp.full((16,), rowi[0], jnp.float32)
                y1i = jnp.full((16,), rowi[1], jnp.float32)
                x2i = jnp.full((16,), rowi[2], jnp.float32)
                y2i = jnp.full((16,), rowi[3], jnp.float32)
                ari = jnp.full((16,), rowi[4], jnp.float32)
                j0 = ((i + 1) // 16) * 16
                nch = (e - j0 + 15) // 16

                def j_body(jb, _2):
                    b0 = j0 + jb * 16
                    pos = b0 + iota
                    oj = gidx_v[pl.ds(b0, 16)]
                    ltx = jnp.maximum(x1i, tabcol(0, oj))
                    lty = jnp.maximum(y1i, tabcol(1, oj))
                    rbx = jnp.minimum(x2i, tabcol(2, oj))
                    rby = jnp.minimum(y2i, tabcol(3, oj))
                    inter = (jnp.maximum(rbx - ltx, 0.0)
                             * jnp.maximum(rby - lty, 0.0))
                    iou = inter / (((ari + tabcol(4, oj)) - inter) + 1e-8)
                    ov = jnp.logical_and(
                        iou > _THR,
                        jnp.logical_and(pos > i, pos < e))
                    old = sup_v[pl.ds(b0, 16)]
                    sup_v[pl.ds(b0, 16)] = jnp.where(ov, 1.0, old)
                    return 0

                lax.fori_loop(0, nch, j_body, 0)

            return 0

        lax.fori_loop(s, e, i_body, 0)

    for k in range(10):
        do_class(slot * 10 + k)

    copies = [
        pltpu.async_copy(sup_v.at[pl.ds(j * 128, 128)],
                         out_hbm.at[oidx_v.at[j]], sem)
        for j in range(_NPAD // 128)
    ]
    for cp in copies:
        cp.wait()


def _sc_nms(tab, gidx, seg):
    mesh = plsc.VectorSubcoreMesh(core_axis_name="c", subcore_axis_name="s")
    f = pl.kernel(
        _sc_body,
        mesh=mesh,
        compiler_params=pltpu.CompilerParams(needs_layout_passes=False),
        out_type=jax.ShapeDtypeStruct((8 * _B * _NPAD,), jnp.float32),
        scratch_types=[
            pltpu.VMEM((_NPAD * 5 + 16,), jnp.float32),
            pltpu.VMEM((_NPAD + 16,), jnp.int32),
            pltpu.VMEM((_SEG + 16,), jnp.int32),
            pltpu.VMEM((_NPAD + 16,), jnp.float32),
            pltpu.VMEM((_NPAD // 128, 128), jnp.int32),
            pltpu.SemaphoreType.DMA,
        ],
    )
    return f(tab.reshape(tab.shape[0], -1), gidx, seg)


def kernel(prediction):
    b, n, _ = prediction.shape
    pred = jnp.pad(prediction, ((0, 0), (0, _NPAD - n), (0, 0)))
    boxes, ext, maxc = _prep(pred)
    maxb = maxc[:, 0:1, 0:1]
    tab, seg3 = _offtab(boxes, ext, maxb)
    seg = seg3[:, 0, :]
    ceff_i = ext[..., 4].astype(jnp.int32)
    gi = lax.broadcasted_iota(jnp.int32, (b, _NPAD), 1)
    _, _, perm = lax.sort((ceff_i, -ext[..., 5], gi), dimension=1,
                          num_keys=2, is_stable=True)
    sup8 = _sc_nms(tab, perm, seg)
    sup = sup8.reshape(8, b, _NPAD).sum(axis=0)
    keep = (1.0 - sup)[:, :n] * ext[:, :n, 3]
    dets = jnp.concatenate([boxes[:, :n, :], ext[:, :n, 0:3]], axis=-1)
    dets = dets * keep[..., None]
    return dets, keep


# speedup vs baseline: 1.0128x; 1.0128x over previous
import jax
import jax.numpy as jnp
from jax import lax
from jax.experimental import pallas as pl
from jax.experimental.pallas import tpu as pltpu
from jax.experimental.pallas import tpu_sc as plsc

_NCLS = 80
_CONF = 0.05
_THR = 0.45
_N = 5000
_NPAD = 5120
_CHUNK = 640
_SEG = 128
_B = 4


def _prep_body(pred_ref, boxes_ref, ext_ref, maxc_ref):
    p = pred_ref[0]
    cx = p[:, 0:1]
    cy = p[:, 1:2]
    w = p[:, 2:3]
    h = p[:, 3:4]
    x1 = cx - w / 2.0
    y1 = cy - h / 2.0
    x2 = cx + w / 2.0
    y2 = cy + h / 2.0
    boxes_ref[0, :, 0:1] = x1
    boxes_ref[0, :, 1:2] = y1
    boxes_ref[0, :, 2:3] = x2
    boxes_ref[0, :, 3:4] = y2
    obj = p[:, 4:5]
    cl = p[:, 5:5 + _NCLS]
    cconf = jnp.max(cl, axis=1, keepdims=True)
    li = lax.broadcasted_iota(jnp.int32, cl.shape, 1)
    cpred = jnp.min(jnp.where(cl == cconf, li, _NCLS), axis=1, keepdims=True)
    score = obj * cconf
    c = pl.program_id(1)
    rid = c * _CHUNK + lax.broadcasted_iota(jnp.int32, (_CHUNK, 1), 0)
    valid = rid < _N
    mask = jnp.logical_and(score >= _CONF, valid)
    ceff = jnp.where(mask, cpred, 127)
    ext_ref[0, :, 0:1] = obj
    ext_ref[0, :, 1:2] = cconf
    ext_ref[0, :, 2:3] = cpred.astype(jnp.float32)
    ext_ref[0, :, 3:4] = mask.astype(jnp.float32)
    ext_ref[0, :, 4:5] = ceff.astype(jnp.float32)
    ext_ref[0, :, 5:6] = score
    zero = jnp.zeros_like(x1)
    m = jnp.maximum(
        jnp.maximum(jnp.max(jnp.where(valid, x1, zero)),
                    jnp.max(jnp.where(valid, y1, zero))),
        jnp.maximum(jnp.max(jnp.where(valid, x2, zero)),
                    jnp.max(jnp.where(valid, y2, zero))))

    @pl.when(c == 0)
    def _():
        maxc_ref[0] = jnp.full((1, 128), m, jnp.float32)

    @pl.when(c > 0)
    def _():
        maxc_ref[0] = jnp.maximum(maxc_ref[0], m)


def _prep(pred):
    b = pred.shape[0]
    nchunks = _NPAD // _CHUNK
    return pl.pallas_call(
        _prep_body,
        grid=(b, nchunks),
        in_specs=[pl.BlockSpec((1, _CHUNK, 5 + _NCLS),
                               lambda i, c: (i, c, 0))],
        out_specs=[
            pl.BlockSpec((1, _CHUNK, 4), lambda i, c: (i, c, 0)),
            pl.BlockSpec((1, _CHUNK, 8), lambda i, c: (i, c, 0)),
            pl.BlockSpec((1, 1, 128), lambda i, c: (i, 0, 0)),
        ],
        out_shape=[
            jax.ShapeDtypeStruct((b, _NPAD, 4), jnp.float32),
            jax.ShapeDtypeStruct((b, _NPAD, 8), jnp.float32),
            jax.ShapeDtypeStruct((b, 1, 128), jnp.float32),
        ],
    )(pred)


def _offtab_body(boxes_ref, ext_ref, maxb_ref, tab_ref, seg_ref):
    mc = maxb_ref[0, 0, 0] + 1.0
    off = ext_ref[0, :, 2:3] * mc
    x1 = boxes_ref[0, :, 0:1] + off
    y1 = boxes_ref[0, :, 1:2] + off
    x2 = boxes_ref[0, :, 2:3] + off
    y2 = boxes_ref[0, :, 3:4] + off
    tab_ref[0, :, 0:1] = x1
    tab_ref[0, :, 1:2] = y1
    tab_ref[0, :, 2:3] = x2
    tab_ref[0, :, 3:4] = y2
    tab_ref[0, :, 4:5] = (jnp.maximum(x2 - x1, 0.0)
                          * jnp.maximum(y2 - y1, 0.0))
    ceff = ext_ref[0, :, 4:5]
    thr = lax.broadcasted_iota(jnp.int32, (1, _SEG), 1).astype(jnp.float32)
    cnt = jnp.sum((ceff < thr).astype(jnp.int32), axis=0, keepdims=True)
    c = pl.program_id(1)

    @pl.when(c == 0)
    def _():
        seg_ref[0] = cnt

    @pl.when(c > 0)
    def _():
        seg_ref[0] = seg_ref[0] + cnt


def _offtab(boxes, ext, maxb):
    b = boxes.shape[0]
    nchunks = _NPAD // _CHUNK
    return pl.pallas_call(
        _offtab_body,
        grid=(b, nchunks),
        in_specs=[
            pl.BlockSpec((1, _CHUNK, 4), lambda i, c: (i, c, 0)),
            pl.BlockSpec((1, _CHUNK, 8), lambda i, c: (i, c, 0)),
            pl.BlockSpec((1, 1, 1), lambda i, c: (i, 0, 0),
                         memory_space=pltpu.SMEM),
        ],
        out_specs=[
            pl.BlockSpec((1, _CHUNK, 5), lambda i, c: (i, c, 0)),
            pl.BlockSpec((1, 1, _SEG), lambda i, c: (i, 0, 0)),
        ],
        out_shape=[
            jax.ShapeDtypeStruct((b, _NPAD, 5), jnp.float32),
            jax.ShapeDtypeStruct((b, 1, _SEG), jnp.int32),
        ],
    )(boxes, ext, maxb)


def _sc_body(tab_hbm, gidx_hbm, seg_hbm, out_hbm,
             tab_v, gidx_v, seg_v, sup_v, oidx_v, sem):
    ci = lax.axis_index("c")
    si = lax.axis_index("s")
    img = ci * 2 + si // 8
    slot = si % 8
    pltpu.sync_copy(tab_hbm.at[img], tab_v.at[pl.ds(0, _NPAD * 5)])
    pltpu.sync_copy(gidx_hbm.at[img], gidx_v.at[pl.ds(0, _NPAD)])
    pltpu.sync_copy(seg_hbm.at[img], seg_v.at[pl.ds(0, _SEG)])
    iota = lax.iota(jnp.int32, 16)
    base_out = (slot * _B + img) * _NPAD

    def init_body(k, _):
        b0 = k * 16
        sup_v[pl.ds(b0, 16)] = jnp.zeros((16,), jnp.float32)
        oidx_v[k // 8, pl.ds((k % 8) * 16, 16)] = (
            gidx_v[pl.ds(b0, 16)] + base_out)
        return 0

    lax.fori_loop(0, _NPAD // 16, init_body, 0)

    def tabcol(col, idx16):
        return plsc.load_gather(tab_v, [idx16 * 5 + col])

    def do_class(c):
        s = seg_v[pl.ds(c, 16)][0]
        e = seg_v[pl.ds(c + 1, 16)][0]

        def i_body(i, _):
            supi = sup_v[pl.ds(i, 16)][0]

            @pl.when(supi < 0.5)
            def _():
                oi = gidx_v[pl.ds(i, 16)][0]
                rowi = tab_v[pl.ds(oi * 5, 16)]
                x1i = jnp.full((16,), rowi[0], jnp.float32)
                y1i = jnp.full((16,), rowi[1], jnp.float32)
                x2i = jnp.full((16,), rowi[2], jnp.float32)
                y2i = jnp.full((16,), rowi[3], jnp.float32)
                ari = jnp.full((16,), rowi[4], jnp.float32)
                j0 = ((i + 1) // 16) * 16
                nch = (e - j0 + 15) // 16

                def j_body(jb, _2):
                    b0 = j0 + jb * 16
                    pos = b0 + iota
                    oj = gidx_v[pl.ds(b0, 16)]
                    ltx = jnp.maximum(x1i, tabcol(0, oj))
                    lty = jnp.maximum(y1i, tabcol(1, oj))
                    rbx = jnp.minimum(x2i, tabcol(2, oj))
                    rby = jnp.minimum(y2i, tabcol(3, oj))
                    inter = (jnp.maximum(rbx - ltx, 0.0)
                             * jnp.maximum(rby - lty, 0.0))
                    iou = inter / (((ari + tabcol(4, oj)) - inter) + 1e-8)
                    ov = jnp.logical_and(
                        iou > _THR,
                        jnp.logical_and(pos > i, pos < e))
                    old = sup_v[pl.ds(b0, 16)]
                    sup_v[pl.ds(b0, 16)] = jnp.where(ov, 1.0, old)
                    return 0

                lax.fori_loop(0, nch, j_body, 0)

            return 0

        lax.fori_loop(s, e, i_body, 0)

    for k in range(10):
        do_class(slot * 10 + k)

    copies = [
        pltpu.async_copy(sup_v.at[pl.ds(j * 128, 128)],
                         out_hbm.at[oidx_v.at[j]], sem)
        for j in range(_NPAD // 128)
    ]
    for cp in copies:
        cp.wait()


def _sc_nms(tab, gidx, seg):
    mesh = plsc.VectorSubcoreMesh(core_axis_name="c", subcore_axis_name="s")
    f = pl.kernel(
        _sc_body,
        mesh=mesh,
        compiler_params=pltpu.CompilerParams(needs_layout_passes=False),
        out_type=jax.ShapeDtypeStruct((8 * _B * _NPAD,), jnp.float32),
        scratch_types=[
            pltpu.VMEM((_NPAD * 5 + 16,), jnp.float32),
            pltpu.VMEM((_NPAD + 16,), jnp.int32),
            pltpu.VMEM((_SEG + 16,), jnp.int32),
            pltpu.VMEM((_NPAD + 16,), jnp.float32),
            pltpu.VMEM((_NPAD // 128, 128), jnp.int32),
            pltpu.SemaphoreType.DMA,
        ],
    )
    return f(tab.reshape(tab.shape[0], -1), gidx, seg)


def kernel(prediction):
    b, n, _ = prediction.shape
    boxes, ext, maxc = _prep(prediction)
    maxb = maxc[:, 0:1, 0:1]
    tab, seg3 = _offtab(boxes, ext, maxb)
    seg = seg3[:, 0, :]
    ceff_i = ext[..., 4].astype(jnp.int32)
    gi = lax.broadcasted_iota(jnp.int32, (b, _NPAD), 1)
    _, _, perm = lax.sort((ceff_i, -ext[..., 5], gi), dimension=1,
                          num_keys=2, is_stable=True)
    sup8 = _sc_nms(tab, perm, seg)
    sup = sup8.reshape(8, b, _NPAD).sum(axis=0)
    keep = (1.0 - sup)[:, :n] * ext[:, :n, 3]
    dets = jnp.concatenate([boxes[:, :n, :], ext[:, :n, 0:3]], axis=-1)
    dets = dets * keep[..., None]
    return dets, keep
